# baseline (speedup 1.0000x reference)
import jax
import jax.numpy as jnp
from jax import lax
from jax.experimental import pallas as pl
from jax.experimental.pallas import tpu as pltpu
from jax.experimental.pallas import tpu_sc as plsc

VOCAB = 100000
DIM = 1024
MAX_SEQ = 8192
B = 4
S = 4096
G = DIM // 128

NC = 2
NS = 16
LANES = 16
NW = NC * NS
NTOK = B * S
ROWS_PER_W = NTOK // NW
CHUNK = 8
NCH = ROWS_PER_W // CHUNK
NBUF = 4
SCH = CHUNK // B


def _body(tok_idx, pos_idx, word_tbl, pos_tbl, out,
          idx_t, idx_p, wbufs, pbufs, sbufs, semw, semp, sems):
    wid = lax.axis_index("s") * NC + lax.axis_index("c")
    s_base = wid * (ROWS_PER_W // B)

    pltpu.sync_copy(tok_idx.at[wid], idx_t)
    pltpu.sync_copy(pos_idx.at[wid], idx_p)

    for bslot in range(NBUF):
        pltpu.async_copy(word_tbl.at[idx_t.at[bslot]], wbufs[bslot], semw[bslot])
        pltpu.async_copy(pos_tbl.at[idx_p.at[bslot]], pbufs[bslot], semp[bslot])

    def outer(g_it, carry):
        for bslot in range(NBUF):
            c = g_it * NBUF + bslot
            wb = wbufs[bslot]
            pb = pbufs[bslot]
            sb = sbufs[bslot]

            pltpu.make_async_copy(word_tbl.at[idx_t.at[c]], wb, semw[bslot]).wait()
            pltpu.make_async_copy(pos_tbl.at[idx_p.at[c]], pb, semp[bslot]).wait()

            @pl.when(c >= NBUF)
            def _():
                pltpu.make_async_copy(
                    sb, out.at[pl.ds(s_base, SCH)], sems[bslot]).wait()

            @plsc.parallel_loop(0, CHUNK, step=1, unroll=4)
            def _add(r):
                si = lax.shift_right_logical(r, 2)
                b = lax.bitwise_and(r, 3)
                for j in range(DIM // LANES):
                    g, lq = j // 8, j % 8
                    sb[si, g, b, pl.ds(lq * LANES, LANES)] = (
                        wb[r, pl.ds(j * LANES, LANES)]
                        + pb[r, pl.ds(j * LANES, LANES)])

            cn = c + NBUF

            @pl.when(cn < NCH)
            def _():
                pltpu.async_copy(word_tbl.at[idx_t.at[cn]], wb, semw[bslot])
                pltpu.async_copy(pos_tbl.at[idx_p.at[cn]], pb, semp[bslot])

            pltpu.async_copy(
                sb, out.at[pl.ds(s_base + c * SCH, SCH)], sems[bslot])
        return carry

    lax.fori_loop(0, NCH // NBUF, outer, 0)

    for bslot in range(NBUF):
        c = NCH - NBUF + bslot
        pltpu.make_async_copy(
            sbufs[bslot], out.at[pl.ds(s_base + c * SCH, SCH)],
            sems[bslot]).wait()


@jax.jit
def _run(tok_idx, pos_idx, word_table, pos_table):
    mesh = plsc.VectorSubcoreMesh(
        core_axis_name="c", subcore_axis_name="s",
        num_cores=NC, num_subcores=NS)
    kfn = pl.kernel(
        _body,
        out_type=jax.ShapeDtypeStruct((S, G, B, 128), jnp.float32),
        mesh=mesh,
        scratch_types=[
            pltpu.VMEM((NCH, CHUNK), jnp.int32),
            pltpu.VMEM((NCH, CHUNK), jnp.int32),
            [pltpu.VMEM((CHUNK, DIM), jnp.float32) for _ in range(NBUF)],
            [pltpu.VMEM((CHUNK, DIM), jnp.float32) for _ in range(NBUF)],
            [pltpu.VMEM((SCH, G, B, 128), jnp.float32) for _ in range(NBUF)],
            [pltpu.SemaphoreType.DMA for _ in range(NBUF)],
            [pltpu.SemaphoreType.DMA for _ in range(NBUF)],
            [pltpu.SemaphoreType.DMA for _ in range(NBUF)],
        ],
    )
    return kfn(tok_idx, pos_idx, word_table, pos_table)


def kernel(tokens, position_ids, word_table, pos_table):
    tok_idx = jnp.transpose(tokens, (1, 0)).astype(jnp.int32).reshape(
        NW, NCH, CHUNK)
    pos_idx = jnp.transpose(position_ids, (1, 0)).astype(jnp.int32).reshape(
        NW, NCH, CHUNK)
    out4 = _run(tok_idx, pos_idx, word_table, pos_table)
    return out4.transpose(0, 2, 1, 3).reshape(S, B, DIM)


# speedup vs baseline: 1.7067x; 1.7067x over previous
import jax
import jax.numpy as jnp
from jax import lax
from jax.experimental import pallas as pl
from jax.experimental.pallas import tpu as pltpu
from jax.experimental.pallas import tpu_sc as plsc

VOCAB = 100000
DIM = 1024
MAX_SEQ = 8192
B = 4
S = 4096
G = DIM // 128

NC = 2
NS = 16
LANES = 16
NW = NC * NS
NTOK = B * S
ROWS_PER_W = NTOK // NW
CHUNK = 8
NCH = ROWS_PER_W // CHUNK
NBUF = 4
SCH = CHUNK // B


def _body(tok_idx, pos_idx, word_tbl, pos_tbl, out,
          idx_t, idx_p, wbufs, pbufs, sbufs, semw, semp, sems):
    wid = lax.axis_index("s") * NC + lax.axis_index("c")
    s_base = wid * (ROWS_PER_W // B)

    pltpu.sync_copy(tok_idx.at[wid], idx_t)
    pltpu.sync_copy(pos_idx.at[wid], idx_p)

    for bslot in range(NBUF):
        pltpu.async_copy(word_tbl.at[idx_t.at[bslot]], wbufs[bslot], semw[bslot])
        pltpu.async_copy(pos_tbl.at[idx_p.at[bslot]], pbufs[bslot], semp[bslot])

    def outer(g_it, carry):
        for bslot in range(NBUF):
            c = g_it * NBUF + bslot
            wb = wbufs[bslot]
            pb = pbufs[bslot]
            sb = sbufs[bslot]

            pltpu.make_async_copy(word_tbl.at[idx_t.at[c]], wb, semw[bslot]).wait()
            pltpu.make_async_copy(pos_tbl.at[idx_p.at[c]], pb, semp[bslot]).wait()

            @pl.when(c >= NBUF)
            def _():
                pltpu.make_async_copy(
                    sb, out.at[pl.ds(s_base, SCH)], sems[bslot]).wait()

            @plsc.parallel_loop(0, CHUNK * G, step=1, unroll=8)
            def _add(it):
                r = lax.shift_right_logical(it, 3)
                g = lax.bitwise_and(it, 7)
                si = lax.shift_right_logical(r, 2)
                b = lax.bitwise_and(r, 3)
                for lq in range(8):
                    sb[si, g, b, pl.ds(lq * LANES, LANES)] = (
                        wb[r, pl.ds(g * 128 + lq * LANES, LANES)]
                        + pb[r, pl.ds(g * 128 + lq * LANES, LANES)])

            cn = c + NBUF

            @pl.when(cn < NCH)
            def _():
                pltpu.async_copy(word_tbl.at[idx_t.at[cn]], wb, semw[bslot])
                pltpu.async_copy(pos_tbl.at[idx_p.at[cn]], pb, semp[bslot])

            pltpu.async_copy(
                sb, out.at[pl.ds(s_base + c * SCH, SCH)], sems[bslot])
        return carry

    lax.fori_loop(0, NCH // NBUF, outer, 0)

    for bslot in range(NBUF):
        c = NCH - NBUF + bslot
        pltpu.make_async_copy(
            sbufs[bslot], out.at[pl.ds(s_base + c * SCH, SCH)],
            sems[bslot]).wait()


@jax.jit
def _run(tok_idx, pos_idx, word_table, pos_table):
    mesh = plsc.VectorSubcoreMesh(
        core_axis_name="c", subcore_axis_name="s",
        num_cores=NC, num_subcores=NS)
    kfn = pl.kernel(
        _body,
        out_type=jax.ShapeDtypeStruct((S, G, B, 128), jnp.float32),
        mesh=mesh,
        scratch_types=[
            pltpu.VMEM((NCH, CHUNK), jnp.int32),
            pltpu.VMEM((NCH, CHUNK), jnp.int32),
            [pltpu.VMEM((CHUNK, DIM), jnp.float32) for _ in range(NBUF)],
            [pltpu.VMEM((CHUNK, DIM), jnp.float32) for _ in range(NBUF)],
            [pltpu.VMEM((SCH, G, B, 128), jnp.float32) for _ in range(NBUF)],
            [pltpu.SemaphoreType.DMA for _ in range(NBUF)],
            [pltpu.SemaphoreType.DMA for _ in range(NBUF)],
            [pltpu.SemaphoreType.DMA for _ in range(NBUF)],
        ],
    )
    return kfn(tok_idx, pos_idx, word_table, pos_table)


def kernel(tokens, position_ids, word_table, pos_table):
    tok_idx = jnp.transpose(tokens, (1, 0)).astype(jnp.int32).reshape(
        NW, NCH, CHUNK)
    pos_idx = jnp.transpose(position_ids, (1, 0)).astype(jnp.int32).reshape(
        NW, NCH, CHUNK)
    out4 = _run(tok_idx, pos_idx, word_table, pos_table)
    return out4.transpose(0, 2, 1, 3).reshape(S, B, DIM)


# speedup vs baseline: 1.7327x; 1.0152x over previous
import jax
import jax.numpy as jnp
from jax import lax
from jax.experimental import pallas as pl
from jax.experimental.pallas import tpu as pltpu
from jax.experimental.pallas import tpu_sc as plsc

VOCAB = 100000
DIM = 1024
MAX_SEQ = 8192
B = 4
S = 4096
G = DIM // 128
LANES = 16

NC = 2
NS = 16
NW = NC * NS
SW = S // NW
SCH = 4
RPC = B * SCH
NCH = SW // SCH
NBUF = 2


def _body(tok_idx, pos_idx, word_tbl, pos_tbl, out,
          idx_t, idx_p, wbufs, pbufs, sbufs, semw, semp, sems):
    wid = lax.axis_index("s") * NC + lax.axis_index("c")
    s0 = wid * SW

    pltpu.sync_copy(tok_idx.at[:, pl.ds(s0, SW)], idx_t)
    pltpu.sync_copy(pos_idx.at[:, pl.ds(s0, SW)], idx_p)

    def fire(c, bslot):
        for bb in range(B):
            pltpu.async_copy(
                word_tbl.at[idx_t.at[bb, pl.ds(c * SCH, SCH)]],
                wbufs[bslot].at[bb], semw[bslot])
            pltpu.async_copy(
                pos_tbl.at[idx_p.at[bb, pl.ds(c * SCH, SCH)]],
                pbufs[bslot].at[bb], semp[bslot])

    for bslot in range(NBUF):
        fire(bslot, bslot)

    def outer(g_it, carry):
        for bslot in range(NBUF):
            c = g_it * NBUF + bslot
            wb = wbufs[bslot]
            pb = pbufs[bslot]
            sb = sbufs[bslot]

            for bb in range(B):
                pltpu.make_async_copy(
                    word_tbl.at[idx_t.at[bb, pl.ds(c * SCH, SCH)]],
                    wb.at[bb], semw[bslot]).wait()
                pltpu.make_async_copy(
                    pos_tbl.at[idx_p.at[bb, pl.ds(c * SCH, SCH)]],
                    pb.at[bb], semp[bslot]).wait()

            @pl.when(c >= NBUF)
            def _():
                pltpu.make_async_copy(
                    sb, out.at[pl.ds(s0, SCH)], sems[bslot]).wait()

            @plsc.parallel_loop(0, RPC * G, step=1, unroll=8)
            def _add(it):
                r = lax.shift_right_logical(it, 3)
                g = lax.bitwise_and(it, 7)
                bb = lax.shift_right_logical(r, 2)
                si = lax.bitwise_and(r, 3)
                for lq in range(8):
                    sb[si, g, bb, pl.ds(lq * LANES, LANES)] = (
                        wb[bb, si, pl.ds(g * 128 + lq * LANES, LANES)]
                        + pb[bb, si, pl.ds(g * 128 + lq * LANES, LANES)])

            cn = c + NBUF

            @pl.when(cn < NCH)
            def _():
                fire(cn, bslot)

            pltpu.async_copy(
                sb, out.at[pl.ds(s0 + c * SCH, SCH)], sems[bslot])
        return carry

    lax.fori_loop(0, NCH // NBUF, outer, 0)

    for bslot in range(NBUF):
        c = NCH - NBUF + bslot
        pltpu.make_async_copy(
            sbufs[bslot], out.at[pl.ds(s0 + c * SCH, SCH)],
            sems[bslot]).wait()


@jax.jit
def _run(tok_idx, pos_idx, word_table, pos_table):
    mesh = plsc.VectorSubcoreMesh(
        core_axis_name="c", subcore_axis_name="s",
        num_cores=NC, num_subcores=NS)
    kfn = pl.kernel(
        _body,
        out_type=jax.ShapeDtypeStruct((S, G, B, 128), jnp.float32),
        mesh=mesh,
        scratch_types=[
            pltpu.VMEM((B, SW), jnp.int32),
            pltpu.VMEM((B, SW), jnp.int32),
            [pltpu.VMEM((B, SCH, DIM), jnp.float32) for _ in range(NBUF)],
            [pltpu.VMEM((B, SCH, DIM), jnp.float32) for _ in range(NBUF)],
            [pltpu.VMEM((SCH, G, B, 128), jnp.float32) for _ in range(NBUF)],
            [pltpu.SemaphoreType.DMA for _ in range(NBUF)],
            [pltpu.SemaphoreType.DMA for _ in range(NBUF)],
            [pltpu.SemaphoreType.DMA for _ in range(NBUF)],
        ],
    )
    return kfn(tok_idx, pos_idx, word_table, pos_table)


def kernel(tokens, position_ids, word_table, pos_table):
    out4 = _run(tokens.astype(jnp.int32), position_ids.astype(jnp.int32),
                word_table, pos_table)
    return out4.transpose(0, 2, 1, 3).reshape(S, B, DIM)


# speedup vs baseline: 1.7444x; 1.0067x over previous
import jax
import jax.numpy as jnp
from jax import lax
from jax.experimental import pallas as pl
from jax.experimental.pallas import tpu as pltpu
from jax.experimental.pallas import tpu_sc as plsc

VOCAB = 100000
DIM = 1024
MAX_SEQ = 8192
B = 4
S = 4096
G = DIM // 128
LANES = 16

NC = 2
NS = 16
NW = NC * NS
SW = S // NW
SCH = 2
RPC = B * SCH
NCH = SW // SCH
NBUF = 4
SCH_BITS = SCH.bit_length() - 1


def _body(tok_idx, pos_idx, word_tbl, pos_tbl, out,
          idx_t, idx_p, wbufs, pbufs, sbufs, semw, semp, sems):
    wid = lax.axis_index("s") * NC + lax.axis_index("c")
    s0 = wid * SW

    pltpu.sync_copy(tok_idx.at[:, pl.ds(s0, SW)], idx_t)
    pltpu.sync_copy(pos_idx.at[:, pl.ds(s0, SW)], idx_p)

    def fire(c, bslot):
        for bb in range(B):
            pltpu.async_copy(
                word_tbl.at[idx_t.at[bb, pl.ds(c * SCH, SCH)]],
                wbufs[bslot].at[bb], semw[bslot])
            pltpu.async_copy(
                pos_tbl.at[idx_p.at[bb, pl.ds(c * SCH, SCH)]],
                pbufs[bslot].at[bb], semp[bslot])

    for bslot in range(NBUF):
        fire(bslot, bslot)

    def outer(g_it, carry):
        for bslot in range(NBUF):
            c = g_it * NBUF + bslot
            wb = wbufs[bslot]
            pb = pbufs[bslot]
            sb = sbufs[bslot]

            for bb in range(B):
                pltpu.make_async_copy(
                    word_tbl.at[idx_t.at[bb, pl.ds(c * SCH, SCH)]],
                    wb.at[bb], semw[bslot]).wait()
                pltpu.make_async_copy(
                    pos_tbl.at[idx_p.at[bb, pl.ds(c * SCH, SCH)]],
                    pb.at[bb], semp[bslot]).wait()

            @pl.when(c >= NBUF)
            def _():
                pltpu.make_async_copy(
                    sb, out.at[pl.ds(s0, SCH)], sems[bslot]).wait()

            @plsc.parallel_loop(0, RPC * G, step=1, unroll=8)
            def _add(it):
                r = lax.shift_right_logical(it, 3)
                g = lax.bitwise_and(it, 7)
                bb = lax.shift_right_logical(r, SCH_BITS)
                si = lax.bitwise_and(r, SCH - 1)
                for lq in range(8):
                    sb[si, g, bb, pl.ds(lq * LANES, LANES)] = (
                        wb[bb, si, pl.ds(g * 128 + lq * LANES, LANES)]
                        + pb[bb, si, pl.ds(g * 128 + lq * LANES, LANES)])

            cn = c + NBUF

            @pl.when(cn < NCH)
            def _():
                fire(cn, bslot)

            pltpu.async_copy(
                sb, out.at[pl.ds(s0 + c * SCH, SCH)], sems[bslot])
        return carry

    lax.fori_loop(0, NCH // NBUF, outer, 0)

    for bslot in range(NBUF):
        c = NCH - NBUF + bslot
        pltpu.make_async_copy(
            sbufs[bslot], out.at[pl.ds(s0 + c * SCH, SCH)],
            sems[bslot]).wait()


@jax.jit
def _run(tok_idx, pos_idx, word_table, pos_table):
    mesh = plsc.VectorSubcoreMesh(
        core_axis_name="c", subcore_axis_name="s",
        num_cores=NC, num_subcores=NS)
    kfn = pl.kernel(
        _body,
        out_type=jax.ShapeDtypeStruct((S, G, B, 128), jnp.float32),
        mesh=mesh,
        scratch_types=[
            pltpu.VMEM((B, SW), jnp.int32),
            pltpu.VMEM((B, SW), jnp.int32),
            [pltpu.VMEM((B, SCH, DIM), jnp.float32) for _ in range(NBUF)],
            [pltpu.VMEM((B, SCH, DIM), jnp.float32) for _ in range(NBUF)],
            [pltpu.VMEM((SCH, G, B, 128), jnp.float32) for _ in range(NBUF)],
            [pltpu.SemaphoreType.DMA for _ in range(NBUF)],
            [pltpu.SemaphoreType.DMA for _ in range(NBUF)],
            [pltpu.SemaphoreType.DMA for _ in range(NBUF)],
        ],
    )
    return kfn(tok_idx, pos_idx, word_table, pos_table)


def kernel(tokens, position_ids, word_table, pos_table):
    out4 = _run(tokens.astype(jnp.int32), position_ids.astype(jnp.int32),
                word_table, pos_table)
    return out4.transpose(0, 2, 1, 3).reshape(S, B, DIM)
